# Initial kernel scaffold; baseline (speedup 1.0000x reference)
import jax
import jax.numpy as jnp
from jax.experimental import pallas as pl


def kernel(h, r_feat, edge_feat, edge_index, invar_ligand_shape, topo_out, e_w, hk_W1, hk_b1, hk_g, hk_be, hk_W2, hk_b2, hv_W1, hv_b1, hv_g, hv_be, hv_W2, hv_b2, hq_W1, hq_b1, hq_g, hq_be, hq_W2, hq_b2, no_W1, no_b1, no_g, no_be, no_W2, no_b2):
    raise NotImplementedError("write your pallas kernel here")


# speedup vs baseline: 5.2542x; 5.2542x over previous
import functools
import math

import jax
import jax.numpy as jnp
from jax import lax
from jax.experimental import pallas as pl
from jax.experimental.pallas import tpu as pltpu
from jax.experimental.pallas import tpu_sc as plsc

NC = 2
NS = 16
GC = 128

LN_EPS = 1e-5


def _ln_relu(x, g, be):
    mu = jnp.mean(x, axis=-1, keepdims=True)
    var = jnp.mean((x - mu) * (x - mu), axis=-1, keepdims=True)
    h = (x - mu) / jnp.sqrt(var + LN_EPS) * g + be
    return jnp.maximum(h, 0.0)


def _head_selector(out_dim, nh):
    head = out_dim // nh
    cc = lax.broadcasted_iota(jnp.int32, (out_dim, nh), 0)
    hh = lax.broadcasted_iota(jnp.int32, (out_dim, nh), 1)
    return (cc // head == hh).astype(jnp.float32)


def _stage1_body(h_ref, topo_ref, ils_ref,
                 wdk, wtk, wik, b1k, wsk,
                 wdv, wtv, wiv, b1v, wsv,
                 qw1, qb1, qg, qbe, qw2, qb2,
                 tdst_ref, tsrc_ref):
    h = h_ref[...]
    t = topo_ref[...]
    s = ils_ref[...]
    ak = h @ wdk[...] + t @ wtk[...] + s @ wik[...] + b1k[...]
    av = h @ wdv[...] + t @ wtv[...] + s @ wiv[...] + b1v[...]
    pre = h @ qw1[...] + qb1[...]
    qq = _ln_relu(pre, qg[...], qbe[...]) @ qw2[...] + qb2[...]
    tdst_ref[...] = jnp.concatenate([ak, av, qq], axis=1)
    tsrc_ref[...] = jnp.concatenate([h @ wsk[...], h @ wsv[...]], axis=1)


def _make_gather(n_nodes, ep):
    GCg = 64
    nchunk = ep // (NC * NS * GCg)
    nh2 = nchunk // 2
    mesh = plsc.VectorSubcoreMesh(core_axis_name="c", subcore_axis_name="s",
                                  num_cores=NC, num_subcores=NS)

    @functools.partial(
        pl.kernel,
        out_type=jax.ShapeDtypeStruct((ep, 384), jnp.float32),
        mesh=mesh,
        scratch_types=[
            pltpu.VMEM((GCg,), jnp.int32),
            pltpu.VMEM((GCg,), jnp.int32),
            pltpu.VMEM((GCg,), jnp.int32),
            pltpu.VMEM((GCg,), jnp.int32),
            pltpu.VMEM((GCg, 384), jnp.float32),
            pltpu.VMEM((GCg, 384), jnp.float32),
            pltpu.VMEM((GCg, 256), jnp.float32),
            pltpu.VMEM((GCg, 256), jnp.float32),
            pltpu.SemaphoreType.DMA,
            pltpu.SemaphoreType.DMA,
            pltpu.SemaphoreType.DMA,
            pltpu.SemaphoreType.DMA,
        ],
    )
    def gather(tdst_hbm, tsrc_hbm, dst_hbm, src_hbm, out_hbm,
               idxd0, idxs0, idxd1, idxs1, gd0, gd1, gs0, gs1,
               sd0, ss0, sd1, ss1):
        c = lax.axis_index("c")
        s = lax.axis_index("s")
        wid = c * NS + s
        cbase = wid * nchunk

        def fire(j, idxd, idxs, gd, gs, sd, ss):
            off = (cbase + j) * GCg
            pltpu.sync_copy(dst_hbm.at[pl.ds(off, GCg)], idxd)
            pltpu.sync_copy(src_hbm.at[pl.ds(off, GCg)], idxs)
            pltpu.async_copy(tdst_hbm.at[idxd], gd, sd)
            pltpu.async_copy(tsrc_hbm.at[idxs], gs, ss)

        def wait(gd, gs, sd, ss):
            pltpu.make_async_copy(tdst_hbm.at[pl.ds(0, GCg), :], gd, sd).wait()
            pltpu.make_async_copy(tsrc_hbm.at[pl.ds(0, GCg), :], gs, ss).wait()

        def drain(j, gd, gs):
            def rowadd(r, cc):
                for i in range(16):
                    sl = pl.ds(i * 16, 16)
                    gd[r, sl] = gd[r, sl] + gs[r, sl]
                return cc

            lax.fori_loop(0, GCg, rowadd, 0)
            off = (cbase + j) * GCg
            pltpu.sync_copy(gd, out_hbm.at[pl.ds(off, GCg), :])

        fire(0, idxd0, idxs0, gd0, gs0, sd0, ss0)

        def body(j2, carry):
            j0 = 2 * j2
            fire(j0 + 1, idxd1, idxs1, gd1, gs1, sd1, ss1)
            wait(gd0, gs0, sd0, ss0)
            drain(j0, gd0, gs0)

            @pl.when(j2 + 1 < nh2)
            def _():
                fire(j0 + 2, idxd0, idxs0, gd0, gs0, sd0, ss0)

            wait(gd1, gs1, sd1, ss1)
            drain(j0 + 1, gd1, gs1)
            return carry

        lax.fori_loop(0, nh2, body, 0)

    return gather


def _stage3_body(g_ref, ef_ref, rf_ref, ew_ref, msk_ref,
                 wek, wrk, gk_g, gk_be, w2k, b2k,
                 wev, wrv, gv_g, gv_be, w2v, b2v,
                 pay_ref):
    g = g_ref[...]
    gk = g[:, 0:128]
    gv = g[:, 128:256]
    qd = g[:, 256:384]
    ef = ef_ref[...]
    rf = rf_ref[...]

    ek = _ln_relu(ef @ wek[...] + rf @ wrk[...] + gk, gk_g[...], gk_be[...])
    k2 = ek @ w2k[...] + b2k[...]
    ev = _ln_relu(ef @ wev[...] + rf @ wrv[...] + gv, gv_g[...], gv_be[...])
    v2 = ev @ w2v[...] + b2v[...]

    sel = _head_selector(128, 16)
    logits = ((qd * k2) @ sel) * (1.0 / math.sqrt(8.0))
    ex = jnp.exp(logits) * msk_ref[...]
    m = (ex @ sel.T) * v2 * ew_ref[...]
    pay_ref[...] = jnp.stack(
        [m[:, k * 16:(k + 1) * 16] for k in range(8)] + [ex], axis=0)


def _make_scatter(n_nodes, ep):
    half = n_nodes // 2
    GCs = 1024
    nchunk = ep // GCs
    mesh = plsc.VectorSubcoreMesh(core_axis_name="c", subcore_axis_name="s",
                                  num_cores=NC, num_subcores=NS)

    @functools.partial(
        pl.kernel,
        out_type=jax.ShapeDtypeStruct((9, n_nodes * 16), jnp.float32),
        mesh=mesh,
        compiler_params=pltpu.CompilerParams(needs_layout_passes=False),
        scratch_types=[
            pltpu.VMEM((GCs,), jnp.int32),
            pltpu.VMEM((GCs * 16,), jnp.float32),
            pltpu.VMEM(((half + 16) * 16,), jnp.float32),
        ],
    )
    def scatter(pay_hbm, idx_hbm, out_hbm, idx_v, pay_v, acc):
        c = lax.axis_index("c")
        s = lax.axis_index("s")
        r = c * NS + s
        a = r // 2
        hh = r % 2

        @pl.when(r < 18)
        def _active():
            zz = jnp.zeros((16,), jnp.float32)

            def zrow(i, carry):
                acc[pl.ds(i * 16, 16)] = zz
                return carry

            lax.fori_loop(0, half + 16, zrow, 0)
            iota = lax.broadcasted_iota(jnp.int32, (16,), 0)
            hbase = hh * half

            def chunk(j, carry):
                off = j * GCs
                pltpu.sync_copy(idx_hbm.at[pl.ds(off, GCs)], idx_v)
                pltpu.sync_copy(pay_hbm.at[a, pl.ds(off * 16, GCs * 16)], pay_v)

                @plsc.parallel_loop(0, GCs // 16, unroll=2)
                def group(g):
                    idxv = idx_v[pl.ds(g * 16, 16)]
                    rel = idxv - hbase
                    okv = (rel >= 0) & (rel < half)
                    relc = jnp.where(okv, rel, half)
                    for l in range(16):
                        lane = jnp.full((16,), l, jnp.int32)
                        rowb = relc.at[lane].get(mode="promise_in_bounds")
                        pos = rowb * 16 + iota
                        val = pay_v[pl.ds((g * 16 + l) * 16, 16)]
                        plsc.addupdate_scatter(acc, [pos], val)

                return carry

            lax.fori_loop(0, nchunk, chunk, 0)
            pltpu.sync_copy(acc.at[pl.ds(0, half * 16)],
                            out_hbm.at[a, pl.ds(hbase * 16, half * 16)])

    return scatter


def _stage5_body(acc_ref, h_ref,
                 nw1, nb1, ng, nbe, nw2, nb2,
                 out_ref):
    num = jnp.concatenate([acc_ref[k] for k in range(8)], axis=1)
    den = acc_ref[8]
    sel = _head_selector(128, 16)
    den_exp = den @ sel.T
    agg = num / (den_exp + 1e-30)
    h = h_ref[...]
    x = jnp.concatenate([agg, h], axis=1)
    pre = x @ nw1[...] + nb1[...]
    out_ref[...] = _ln_relu(pre, ng[...], nbe[...]) @ nw2[...] + nb2[...] + h


def _full(block, ndim_idx):
    return pl.BlockSpec(block, lambda *args: tuple(0 for _ in range(ndim_idx)))


def kernel(h, r_feat, edge_feat, edge_index, invar_ligand_shape, topo_out, e_w,
           hk_W1, hk_b1, hk_g, hk_be, hk_W2, hk_b2,
           hv_W1, hv_b1, hv_g, hv_be, hv_W2, hv_b2,
           hq_W1, hq_b1, hq_g, hq_be, hq_W2, hq_b2,
           no_W1, no_b1, no_g, no_be, no_W2, no_b2):
    N, D = h.shape
    E = r_feat.shape[0]
    EF = edge_feat.shape[1]
    SD = invar_ligand_shape.shape[1]
    HID = topo_out.shape[1]
    RF = r_feat.shape[1]

    def row(b):
        return b.reshape(1, -1)

    o_ef, o_rf, o_hd, o_hs, o_tp, o_il = (
        0, EF, EF + RF, EF + RF + D, EF + RF + 2 * D, EF + RF + 2 * D + HID)
    parts = {}
    for name, W1, b1 in (("k", hk_W1, hk_b1), ("v", hv_W1, hv_b1)):
        parts[name] = dict(
            we=W1[o_ef:o_rf], wr=W1[o_rf:o_hd], wd=W1[o_hd:o_hs],
            ws=W1[o_hs:o_tp], wt=W1[o_tp:o_il], wi=W1[o_il:], b1=row(b1))

    WCHUNK = NC * NS * GC
    Ep = ((E + WCHUNK - 1) // WCHUNK) * WCHUNK
    pad = Ep - E
    src = edge_index[0].astype(jnp.int32)
    dst = edge_index[1].astype(jnp.int32)
    dstp = jnp.concatenate([dst, jnp.zeros((pad,), jnp.int32)])
    srcp = jnp.concatenate([src, jnp.zeros((pad,), jnp.int32)])
    efp = jnp.pad(edge_feat, ((0, pad), (0, 0)))
    rfp = jnp.pad(r_feat, ((0, pad), (0, 0)))
    ewp = jnp.pad(e_w, (0, pad)).reshape(Ep, 1)
    maskp = jnp.pad(jnp.ones((E,), jnp.float32), (0, pad)).reshape(Ep, 1)
    N2 = ((N + NS * 8 - 1) // (NS * 8)) * (NS * 8)

    BN = 1000
    wspec = lambda shp: pl.BlockSpec(shp, lambda i: (0,) * len(shp))
    tdst, tsrc = pl.pallas_call(
        _stage1_body,
        grid=(N // BN,),
        in_specs=[
            pl.BlockSpec((BN, D), lambda i: (i, 0)),
            pl.BlockSpec((BN, HID), lambda i: (i, 0)),
            pl.BlockSpec((BN, SD), lambda i: (i, 0)),
            wspec((D, HID)), wspec((HID, HID)), wspec((SD, HID)),
            wspec((1, HID)), wspec((D, HID)),
            wspec((D, HID)), wspec((HID, HID)), wspec((SD, HID)),
            wspec((1, HID)), wspec((D, HID)),
            wspec((D, HID)), wspec((1, HID)), wspec((1, HID)),
            wspec((1, HID)), wspec((HID, 128)), wspec((1, 128)),
        ],
        out_specs=[
            pl.BlockSpec((BN, 384), lambda i: (i, 0)),
            pl.BlockSpec((BN, 256), lambda i: (i, 0)),
        ],
        out_shape=[
            jax.ShapeDtypeStruct((N, 384), jnp.float32),
            jax.ShapeDtypeStruct((N, 256), jnp.float32),
        ],
    )(h, topo_out, invar_ligand_shape,
      parts["k"]["wd"], parts["k"]["wt"], parts["k"]["wi"], parts["k"]["b1"],
      parts["k"]["ws"],
      parts["v"]["wd"], parts["v"]["wt"], parts["v"]["wi"], parts["v"]["b1"],
      parts["v"]["ws"],
      hq_W1, row(hq_b1), row(hq_g), row(hq_be), hq_W2, row(hq_b2))

    g = _make_gather(N, Ep)(tdst, tsrc, dstp, srcp)

    BE = 2048
    pay = pl.pallas_call(
        _stage3_body,
        grid=(Ep // BE,),
        in_specs=[
            pl.BlockSpec((BE, 384), lambda i: (i, 0)),
            pl.BlockSpec((BE, EF), lambda i: (i, 0)),
            pl.BlockSpec((BE, RF), lambda i: (i, 0)),
            pl.BlockSpec((BE, 1), lambda i: (i, 0)),
            pl.BlockSpec((BE, 1), lambda i: (i, 0)),
            wspec((EF, HID)), wspec((RF, HID)), wspec((1, HID)),
            wspec((1, HID)), wspec((HID, 128)), wspec((1, 128)),
            wspec((EF, HID)), wspec((RF, HID)), wspec((1, HID)),
            wspec((1, HID)), wspec((HID, 128)), wspec((1, 128)),
        ],
        out_specs=pl.BlockSpec((9, BE, 16), lambda i: (0, i, 0)),
        out_shape=jax.ShapeDtypeStruct((9, Ep, 16), jnp.float32),
    )(g, efp, rfp, ewp, maskp,
      parts["k"]["we"], parts["k"]["wr"], row(hk_g), row(hk_be), hk_W2, row(hk_b2),
      parts["v"]["we"], parts["v"]["wr"], row(hv_g), row(hv_be), hv_W2, row(hv_b2))

    acc = _make_scatter(N2, Ep)(pay.reshape(9, Ep * 16), dstp)
    accn = acc.reshape(9, N2, 16)[:, :N, :]

    out = pl.pallas_call(
        _stage5_body,
        grid=(N // BN,),
        in_specs=[
            pl.BlockSpec((9, BN, 16), lambda i: (0, i, 0)),
            pl.BlockSpec((BN, D), lambda i: (i, 0)),
            wspec((2 * HID, HID)), wspec((1, HID)), wspec((1, HID)),
            wspec((1, HID)), wspec((HID, HID)), wspec((1, HID)),
        ],
        out_specs=pl.BlockSpec((BN, HID), lambda i: (i, 0)),
        out_shape=jax.ShapeDtypeStruct((N, HID), jnp.float32),
    )(accn, h, no_W1, row(no_b1), row(no_g), row(no_be), no_W2, row(no_b2))
    return out
